# Initial kernel scaffold; baseline (speedup 1.0000x reference)
import jax
import jax.numpy as jnp
from jax.experimental import pallas as pl


def kernel(data, t0, tn, beta, z0, v0, a0, pair_u, pair_v):
    raise NotImplementedError("write your pallas kernel here")


# speedup vs baseline: 18.2769x; 18.2769x over previous
import jax
import jax.numpy as jnp
from jax import lax
from jax.experimental import pallas as pl
from jax.experimental.pallas import tpu as pltpu
from jax.experimental.pallas import tpu_sc as plsc

_NC = 2
_NS = 16
_L = 16
_NW = _NC * _NS

_E = 50000
_EPW = 1568
_EP = _NW * _EPW
_ESTEPS = _EPW // _L

_S = 2000
_SPW = 64
_SP = _NW * _SPW
_PSTEPS = _SPW // _L

_R = 10
_N = 5000
_EPS = 1e-6


def _rsqrt(x):
  xh = x * 0.5
  i = plsc.bitcast(x, jnp.int32)
  i = 0x5F3759DF - (i >> 1)
  y = plsc.bitcast(i, jnp.float32)
  y = y * (1.5 - xh * y * y)
  y = y * (1.5 - xh * y * y)
  y = y * (1.5 - xh * y * y)
  return y


def _sc_body(tab_h, ev_h, pr_h, cst_h, o_h,
             tab_v, ev_v, pr_v, cst_v, acc_v, sem):
  cid = lax.axis_index("c")
  sid = lax.axis_index("s")
  wid = sid * _NC + cid

  copies = [
      pltpu.async_copy(tab_h, tab_v, sem),
      pltpu.async_copy(ev_h.at[wid], ev_v, sem),
      pltpu.async_copy(pr_h.at[wid], pr_v, sem),
      pltpu.async_copy(cst_h, cst_v, sem),
  ]
  for cp in copies:
    cp.wait()

  rows = [jnp.full((_L,), r, jnp.int32) for r in range(6)]

  def gather(r, idx):
    return plsc.load_gather(tab_v, [rows[r], idx])

  def pair_diffs(u, v):
    dzx = gather(0, u) - gather(0, v)
    dzy = gather(1, u) - gather(1, v)
    dvx = gather(2, u) - gather(2, v)
    dvy = gather(3, u) - gather(3, v)
    dax = gather(4, u) - gather(4, v)
    day = gather(5, u) - gather(5, v)
    return dzx, dzy, dvx, dvy, dax, day

  def dist(diffs, t):
    dzx, dzy, dvx, dvy, dax, day = diffs
    t2h = t * t * 0.5
    px = dzx + dvx * t + dax * t2h + _EPS
    py = dzy + dvy * t + day * t2h + _EPS
    d2 = px * px + py * py
    return d2 * _rsqrt(d2)

  def ebody(i, acc):
    b = i * _L
    u = ev_v[pl.ds(b, _L)].astype(jnp.int32)
    v = ev_v[pl.ds(_EPW + b, _L)].astype(jnp.int32)
    t = ev_v[pl.ds(2 * _EPW + b, _L)]
    m = ev_v[pl.ds(3 * _EPW + b, _L)]
    d = dist(pair_diffs(u, v), t)
    return acc + d * m

  acc_e = lax.fori_loop(0, _ESTEPS, ebody, jnp.zeros((_L,), jnp.float32))

  beta = cst_v[pl.ds(0, _L)]

  def pbody(i, acc):
    b = i * _L
    pu = pr_v[pl.ds(b, _L)].astype(jnp.int32)
    pv = pr_v[pl.ds(_SPW + b, _L)].astype(jnp.int32)
    pm = pr_v[pl.ds(2 * _SPW + b, _L)]
    diffs = pair_diffs(pu, pv)
    for j in range(_R):
      tj = cst_v[pl.ds(_L + j * _L, _L)]
      d = dist(diffs, tj)
      acc = acc + jnp.exp(beta - d) * pm
    return acc

  acc_n = lax.fori_loop(0, _PSTEPS, pbody, jnp.zeros((_L,), jnp.float32))

  acc_v[pl.ds(0, _L)] = acc_e
  acc_v[pl.ds(_L, _L)] = acc_n
  pltpu.sync_copy(acc_v, o_h.at[pl.ds(wid * 2 * _L, 2 * _L)])


@jax.jit
def _sc_call(tab, ev, pr, cst):
  mesh = plsc.VectorSubcoreMesh(
      core_axis_name="c", subcore_axis_name="s",
      num_cores=_NC, num_subcores=_NS)
  f = pl.kernel(
      _sc_body,
      out_type=jax.ShapeDtypeStruct((_NW * 2 * _L,), jnp.float32),
      mesh=mesh,
      compiler_params=pltpu.CompilerParams(needs_layout_passes=False),
      scratch_types=[
          pltpu.VMEM((6, _N), jnp.float32),
          pltpu.VMEM((4 * _EPW,), jnp.float32),
          pltpu.VMEM((3 * _SPW,), jnp.float32),
          pltpu.VMEM((_L + _R * _L,), jnp.float32),
          pltpu.VMEM((2 * _L,), jnp.float32),
          pltpu.SemaphoreType.DMA,
      ],
  )
  return f(tab, ev, pr, cst)


def _tc_body(evp_ref, nep_ref, beta_ref, dx_ref, out_ref):
  ev = jnp.sum(evp_ref[...])
  ne = jnp.sum(nep_ref[...])
  out_ref[0, 0] = _E * beta_ref[0, 0] - ev - dx_ref[0, 0] * ne


@jax.jit
def _tc_call(evp, nep, beta, dx):
  return pl.pallas_call(
      _tc_body,
      out_shape=jax.ShapeDtypeStruct((1, 1), jnp.float32),
      out_specs=pl.BlockSpec(memory_space=pltpu.SMEM),
      in_specs=[
          pl.BlockSpec(memory_space=pltpu.VMEM),
          pl.BlockSpec(memory_space=pltpu.VMEM),
          pl.BlockSpec(memory_space=pltpu.SMEM),
          pl.BlockSpec(memory_space=pltpu.SMEM),
      ],
  )(evp, nep, beta, dx)


def kernel(data, t0, tn, beta, z0, v0, a0, pair_u, pair_v):
  e = data.shape[0]
  s = pair_u.shape[0]

  u = data[:, 0]
  v = data[:, 1]
  t = data[:, 2]
  pad_e = _EP - e
  zpe = jnp.zeros((pad_e,), jnp.float32)
  ev = jnp.concatenate([
      jnp.concatenate([u, zpe]).reshape(_NW, _EPW),
      jnp.concatenate([v, zpe]).reshape(_NW, _EPW),
      jnp.concatenate([t, zpe]).reshape(_NW, _EPW),
      jnp.concatenate([jnp.ones((e,), jnp.float32), zpe]).reshape(_NW, _EPW),
  ], axis=1)

  pad_s = _SP - s
  zps = jnp.zeros((pad_s,), jnp.float32)
  pr = jnp.concatenate([
      jnp.concatenate([pair_u.astype(jnp.float32), zps]).reshape(_NW, _SPW),
      jnp.concatenate([pair_v.astype(jnp.float32), zps]).reshape(_NW, _SPW),
      jnp.concatenate([jnp.ones((s,), jnp.float32), zps]).reshape(_NW, _SPW),
  ], axis=1)

  tab = jnp.concatenate([z0.T, v0.T, a0.T], axis=0)

  t0s = t0[0]
  tns = tn[0]
  x = t0s + (tns - t0s) * jnp.arange(_R + 1, dtype=jnp.float32) / _R
  xm = (x[:-1] + x[1:]) * 0.5
  cst = jnp.concatenate([
      jnp.full((_L,), beta[0, 0], jnp.float32),
      jnp.broadcast_to(xm[:, None], (_R, _L)).reshape(-1),
  ])

  dx = ((tns - t0s) / _R).reshape(1, 1)

  parts = _sc_call(tab, ev, pr, cst).reshape(_NW, 2, _L)
  return _tc_call(parts[:, 0], parts[:, 1], beta, dx)
